# baseline (speedup 1.0000x reference)
import jax
import jax.numpy as jnp
from jax import lax
from jax.experimental import pallas as pl
from jax.experimental.pallas import tpu as pltpu
from jax.experimental.pallas import tpu_sc as plsc

VOCAB = 1_000_000
D = 64
DPAD = 128
B_TOTAL = 4096 * 200
NC, NS = 2, 16
NW = NC * NS
PER_W = B_TOTAL // NW
CHUNK = 128
NCH = PER_W // CHUNK
NB = 3
NITER = (NCH + NB - 1) // NB
SCALE = 8.0

TCB = 512
TC_GRID = (VOCAB + TCB - 1) // TCB


def _tc_body(tt_ref, out_ref):
  r = lax.broadcasted_iota(jnp.int32, (D, DPAD), 0)
  c = lax.broadcasted_iota(jnp.int32, (D, DPAD), 1)
  ident2 = jnp.where(r == c % D, 1.0, 0.0).astype(jnp.float32)
  out_ref[...] = lax.dot_general(
      tt_ref[...], ident2, (((0,), (0,)), ((), ())),
      preferred_element_type=jnp.float32)


def _sc_body(table_ref, idx_ref, out_ref,
             idxv,
             gb0, gb1, gb2,
             sb0, sb1, sb2,
             gs0, gs1, gs2,
             os0, os1, os2):
  gb = [gb0, gb1, gb2]
  sb = [sb0, sb1, sb2]
  gsem = [gs0, gs1, gs2]
  osem = [os0, os1, os2]

  wid = lax.axis_index("s") * NC + lax.axis_index("c")
  base_row = wid * NCH
  base_out = wid * PER_W

  pltpu.sync_copy(idx_ref.at[pl.ds(base_row, NCH)], idxv)

  for b in range(NB):
    pltpu.async_copy(table_ref.at[idxv.at[b]], gb[b], gsem[b])

  def outer(g, carry):
    for b in range(NB):
      j = g * NB + b

      @pl.when(j < NCH)
      def _():
        pltpu.make_async_copy(table_ref.at[idxv.at[j]], gb[b], gsem[b]).wait()

        @pl.when(j >= NB)
        def _():
          pltpu.make_async_copy(
              sb[b], out_ref.at[pl.ds(base_out, CHUNK)], osem[b]).wait()

        def scale_row(r, c2):
          for c in range(D // 16):
            sb[b][r, pl.ds(c * 16, 16)] = gb[b][r, pl.ds(c * 16, 16)] * SCALE
          return c2
        lax.fori_loop(0, CHUNK, scale_row, 0, unroll=4)

        pltpu.async_copy(
            sb[b], out_ref.at[pl.ds(base_out + j * CHUNK, CHUNK)], osem[b])

        @pl.when(j + NB < NCH)
        def _():
          pltpu.async_copy(table_ref.at[idxv.at[j + NB]], gb[b], gsem[b])
    return carry

  lax.fori_loop(0, NITER, outer, 0)

  for b in range(NB):
    pltpu.make_async_copy(
        sb[b], out_ref.at[pl.ds(base_out, CHUNK)], osem[b]).wait()


def kernel(x, table):
  idx = x.astype(jnp.int32).reshape(NW * NCH, CHUNK)

  tablep = pl.pallas_call(
      _tc_body,
      grid=(TC_GRID,),
      in_specs=[pl.BlockSpec((D, TCB), lambda i: (0, i))],
      out_specs=pl.BlockSpec((TCB, DPAD), lambda i: (i, 0)),
      out_shape=jax.ShapeDtypeStruct((TC_GRID * TCB, DPAD), jnp.float32),
  )(table.T)

  mesh = plsc.VectorSubcoreMesh(core_axis_name="c", subcore_axis_name="s")
  k = pl.kernel(
      _sc_body,
      mesh=mesh,
      compiler_params=pltpu.CompilerParams(use_tc_tiling_on_sc=True),
      out_type=jax.ShapeDtypeStruct((B_TOTAL, D), jnp.float32),
      scratch_types=(
          [pltpu.VMEM((NCH, CHUNK), jnp.int32)]
          + [pltpu.VMEM((CHUNK, DPAD), jnp.float32) for _ in range(NB)]
          + [pltpu.VMEM((CHUNK, D), jnp.float32) for _ in range(NB)]
          + [pltpu.SemaphoreType.DMA for _ in range(2 * NB)]
      ),
  )
  out = k(tablep, idx)
  return out.reshape(4096, 200, D)


# speedup vs baseline: 1.9137x; 1.9137x over previous
import jax
import jax.numpy as jnp
from jax import lax
from jax.experimental import pallas as pl
from jax.experimental.pallas import tpu as pltpu
from jax.experimental.pallas import tpu_sc as plsc

VOCAB = 1_000_000
D = 64
DPAD = 128
B_TOTAL = 4096 * 200
NC, NS = 2, 16
NW = NC * NS
PER_W = B_TOTAL // NW
CHUNK = 128
NCH = PER_W // CHUNK
NB = 3
NITER = (NCH + NB - 1) // NB
SCALE = 8.0

TCB = 4096
TC_GRID = (VOCAB + TCB - 1) // TCB


def _tc_body(tt_ref, out_ref):
  r = lax.broadcasted_iota(jnp.int32, (D, DPAD), 0)
  c = lax.broadcasted_iota(jnp.int32, (D, DPAD), 1)
  ident2 = jnp.where(r == c % D, 1.0, 0.0).astype(jnp.float32)
  out_ref[...] = lax.dot_general(
      tt_ref[...], ident2, (((0,), (0,)), ((), ())),
      preferred_element_type=jnp.float32)


def _sc_body(table_ref, idx_ref, out_ref,
             idxv,
             gb0, gb1, gb2,
             sb0, sb1, sb2,
             gs0, gs1, gs2,
             os0, os1, os2):
  gb = [gb0, gb1, gb2]
  sb = [sb0, sb1, sb2]
  gsem = [gs0, gs1, gs2]
  osem = [os0, os1, os2]

  wid = lax.axis_index("s") * NC + lax.axis_index("c")
  base_row = wid * NCH
  base_out = wid * PER_W

  pltpu.sync_copy(idx_ref.at[pl.ds(base_row, NCH)], idxv)

  for b in range(NB):
    pltpu.async_copy(table_ref.at[idxv.at[b]], gb[b], gsem[b])

  def outer(g, carry):
    for b in range(NB):
      j = g * NB + b

      @pl.when(j < NCH)
      def _():
        pltpu.make_async_copy(table_ref.at[idxv.at[j]], gb[b], gsem[b]).wait()

        @pl.when(j >= NB)
        def _():
          pltpu.make_async_copy(
              sb[b], out_ref.at[pl.ds(base_out, CHUNK)], osem[b]).wait()

        def scale_row(r, c2):
          for c in range(D // 16):
            sb[b][r, pl.ds(c * 16, 16)] = gb[b][r, pl.ds(c * 16, 16)] * SCALE
          return c2
        lax.fori_loop(0, CHUNK, scale_row, 0, unroll=4)

        pltpu.async_copy(
            sb[b], out_ref.at[pl.ds(base_out + j * CHUNK, CHUNK)], osem[b])

        @pl.when(j + NB < NCH)
        def _():
          pltpu.async_copy(table_ref.at[idxv.at[j + NB]], gb[b], gsem[b])
    return carry

  lax.fori_loop(0, NITER, outer, 0)

  for b in range(NB):
    pltpu.make_async_copy(
        sb[b], out_ref.at[pl.ds(base_out, CHUNK)], osem[b]).wait()


def kernel(x, table):
  idx = x.astype(jnp.int32).reshape(NW * NCH, CHUNK)

  tablep = pl.pallas_call(
      _tc_body,
      grid=(TC_GRID,),
      in_specs=[pl.BlockSpec((D, TCB), lambda i: (0, i))],
      out_specs=pl.BlockSpec((TCB, DPAD), lambda i: (i, 0)),
      out_shape=jax.ShapeDtypeStruct((TC_GRID * TCB, DPAD), jnp.float32),
  )(table.T)

  mesh = plsc.VectorSubcoreMesh(core_axis_name="c", subcore_axis_name="s")
  k = pl.kernel(
      _sc_body,
      mesh=mesh,
      compiler_params=pltpu.CompilerParams(use_tc_tiling_on_sc=True),
      out_type=jax.ShapeDtypeStruct((B_TOTAL, D), jnp.float32),
      scratch_types=(
          [pltpu.VMEM((NCH, CHUNK), jnp.int32)]
          + [pltpu.VMEM((CHUNK, DPAD), jnp.float32) for _ in range(NB)]
          + [pltpu.VMEM((CHUNK, D), jnp.float32) for _ in range(NB)]
          + [pltpu.SemaphoreType.DMA for _ in range(2 * NB)]
      ),
  )
  out = k(tablep, idx)
  return out.reshape(4096, 200, D)


# speedup vs baseline: 2.5661x; 1.3409x over previous
import jax
import jax.numpy as jnp
from jax import lax
from jax.experimental import pallas as pl
from jax.experimental.pallas import tpu as pltpu
from jax.experimental.pallas import tpu_sc as plsc

VOCAB = 1_000_000
D = 64
DPAD = 128
B_TOTAL = 4096 * 200
NC, NS = 2, 16
NW = NC * NS
PER_W = B_TOTAL // NW
CHUNK = 128
NCH = PER_W // CHUNK
NB = 6
NITER = (NCH + NB - 1) // NB
SCALE = 8.0

TCB = 8192
TC_GRID = (VOCAB + TCB - 1) // TCB


def _tc_body(tt_ref, out_ref):
  r = lax.broadcasted_iota(jnp.int32, (D, DPAD), 0)
  c = lax.broadcasted_iota(jnp.int32, (D, DPAD), 1)
  ident2 = jnp.where(r == c % D, SCALE, 0.0).astype(jnp.float32)
  out_ref[...] = lax.dot_general(
      tt_ref[...], ident2, (((0,), (0,)), ((), ())),
      preferred_element_type=jnp.float32)


def _sc_body(table_ref, idx_ref, out_ref,
             idxv,
             gb0, gb1, gb2, gb3, gb4, gb5,
             gs0, gs1, gs2, gs3, gs4, gs5,
             os0, os1, os2, os3, os4, os5):
  gb = [gb0, gb1, gb2, gb3, gb4, gb5]
  gsem = [gs0, gs1, gs2, gs3, gs4, gs5]
  osem = [os0, os1, os2, os3, os4, os5]

  wid = lax.axis_index("s") * NC + lax.axis_index("c")
  base_row = wid * NCH
  base_out = wid * PER_W

  pltpu.sync_copy(idx_ref.at[pl.ds(base_row, NCH)], idxv)

  for b in range(NB):
    pltpu.async_copy(table_ref.at[idxv.at[b]], gb[b], gsem[b])

  def outer(g, carry):
    for b in range(NB):
      j = g * NB + b

      @pl.when(j < NCH)
      def _():
        pltpu.make_async_copy(table_ref.at[idxv.at[j]], gb[b], gsem[b]).wait()

        pltpu.async_copy(
            gb[b], out_ref.at[pl.ds(base_out + j * CHUNK, CHUNK)], osem[b])

        @pl.when(j + NB < NCH)
        def _():
          pltpu.make_async_copy(
              gb[b], out_ref.at[pl.ds(base_out, CHUNK)], osem[b]).wait()
          pltpu.async_copy(table_ref.at[idxv.at[j + NB]], gb[b], gsem[b])
    return carry

  lax.fori_loop(0, NITER, outer, 0)

  for b in range(NB):
    pltpu.make_async_copy(
        gb[b], out_ref.at[pl.ds(base_out, CHUNK)], osem[b]).wait()


def kernel(x, table):
  idx = x.astype(jnp.int32).reshape(NW * NCH, CHUNK)

  tablep = pl.pallas_call(
      _tc_body,
      grid=(TC_GRID,),
      in_specs=[pl.BlockSpec((D, TCB), lambda i: (0, i))],
      out_specs=pl.BlockSpec((TCB, DPAD), lambda i: (i, 0)),
      out_shape=jax.ShapeDtypeStruct((TC_GRID * TCB, DPAD), jnp.float32),
  )(table.T)

  mesh = plsc.VectorSubcoreMesh(core_axis_name="c", subcore_axis_name="s")
  k = pl.kernel(
      _sc_body,
      mesh=mesh,
      compiler_params=pltpu.CompilerParams(use_tc_tiling_on_sc=True),
      out_type=jax.ShapeDtypeStruct((B_TOTAL, DPAD), jnp.float32),
      scratch_types=(
          [pltpu.VMEM((NCH, CHUNK), jnp.int32)]
          + [pltpu.VMEM((CHUNK, DPAD), jnp.float32) for _ in range(NB)]
          + [pltpu.SemaphoreType.DMA for _ in range(2 * NB)]
      ),
  )
  out = k(tablep, idx)
  return out[:, :D].reshape(4096, 200, D)
